# Initial kernel scaffold; baseline (speedup 1.0000x reference)
import jax
import jax.numpy as jnp
from jax.experimental import pallas as pl


def kernel(x, W, gamma, beta):
    raise NotImplementedError("write your pallas kernel here")


# speedup vs baseline: 3.2567x; 3.2567x over previous
import jax
import jax.numpy as jnp
from jax.experimental import pallas as pl
from jax.experimental.pallas import tpu as pltpu

_BLK = 512


def _ln_kernel(x_ref, w_ref, g_ref, b_ref, o_ref):
    x = x_ref[0]
    e = w_ref[0, 0]
    y = x + e[None, :]
    mean = jnp.mean(y, axis=1, keepdims=True)
    yc = y - mean
    var = jnp.mean(yc * yc, axis=1, keepdims=True)
    inv = jax.lax.rsqrt(var + 1e-9)
    o_ref[0] = yc * inv * g_ref[0][None, :] + b_ref[0][None, :]


def kernel(x, W, gamma, beta):
    B, S, D = x.shape
    W3 = W.reshape(W.shape[0], 1, D)
    g2 = gamma.reshape(1, D)
    b2 = beta.reshape(1, D)
    grid = (B, S // _BLK)
    return pl.pallas_call(
        _ln_kernel,
        grid=grid,
        in_specs=[
            pl.BlockSpec((1, _BLK, D), lambda b, s: (b, s, 0)),
            pl.BlockSpec((1, 1, D), lambda b, s: (b, 0, 0)),
            pl.BlockSpec((1, D), lambda b, s: (0, 0)),
            pl.BlockSpec((1, D), lambda b, s: (0, 0)),
        ],
        out_specs=pl.BlockSpec((1, _BLK, D), lambda b, s: (b, s, 0)),
        out_shape=jax.ShapeDtypeStruct((B, S, D), x.dtype),
        compiler_params=pltpu.CompilerParams(
            dimension_semantics=("parallel", "parallel"),
        ),
    )(x, W3, g2, b2)


# speedup vs baseline: 3.6877x; 1.1324x over previous
import jax
import jax.numpy as jnp
from jax.experimental import pallas as pl
from jax.experimental.pallas import tpu as pltpu

_BLK = 1024


def _ln_kernel(x_ref, w_ref, g_ref, b_ref, o_ref):
    x = x_ref[0]
    e = w_ref[0, 0]
    y = x + e[None, :]
    mean = jnp.mean(y, axis=1, keepdims=True)
    yc = y - mean
    var = jnp.mean(yc * yc, axis=1, keepdims=True)
    inv = jax.lax.rsqrt(var + 1e-9)
    o_ref[0] = yc * inv * g_ref[0][None, :] + b_ref[0][None, :]


def kernel(x, W, gamma, beta):
    B, S, D = x.shape
    W3 = W.reshape(W.shape[0], 1, D)
    g2 = gamma.reshape(1, D)
    b2 = beta.reshape(1, D)
    grid = (B, S // _BLK)
    return pl.pallas_call(
        _ln_kernel,
        grid=grid,
        in_specs=[
            pl.BlockSpec((1, _BLK, D), lambda b, s: (b, s, 0)),
            pl.BlockSpec((1, 1, D), lambda b, s: (b, 0, 0)),
            pl.BlockSpec((1, D), lambda b, s: (0, 0)),
            pl.BlockSpec((1, D), lambda b, s: (0, 0)),
        ],
        out_specs=pl.BlockSpec((1, _BLK, D), lambda b, s: (b, s, 0)),
        out_shape=jax.ShapeDtypeStruct((B, S, D), x.dtype),
        compiler_params=pltpu.CompilerParams(
            dimension_semantics=("parallel", "parallel"),
        ),
    )(x, W3, g2, b2)


# speedup vs baseline: 3.7902x; 1.0278x over previous
import jax
import jax.numpy as jnp
from jax.experimental import pallas as pl
from jax.experimental.pallas import tpu as pltpu

_BLK = 2048


def _ln_kernel(x_ref, w_ref, g_ref, b_ref, o_ref):
    x = x_ref[0]
    e = w_ref[0, 0]
    y = x + e[None, :]
    mean = jnp.mean(y, axis=1, keepdims=True)
    yc = y - mean
    var = jnp.mean(yc * yc, axis=1, keepdims=True)
    inv = jax.lax.rsqrt(var + 1e-9)
    o_ref[0] = yc * inv * g_ref[0][None, :] + b_ref[0][None, :]


def kernel(x, W, gamma, beta):
    B, S, D = x.shape
    W3 = W.reshape(W.shape[0], 1, D)
    g2 = gamma.reshape(1, D)
    b2 = beta.reshape(1, D)
    grid = (B, S // _BLK)
    return pl.pallas_call(
        _ln_kernel,
        grid=grid,
        in_specs=[
            pl.BlockSpec((1, _BLK, D), lambda b, s: (b, s, 0)),
            pl.BlockSpec((1, 1, D), lambda b, s: (b, 0, 0)),
            pl.BlockSpec((1, D), lambda b, s: (0, 0)),
            pl.BlockSpec((1, D), lambda b, s: (0, 0)),
        ],
        out_specs=pl.BlockSpec((1, _BLK, D), lambda b, s: (b, s, 0)),
        out_shape=jax.ShapeDtypeStruct((B, S, D), x.dtype),
        compiler_params=pltpu.CompilerParams(
            dimension_semantics=("parallel", "parallel"),
        ),
    )(x, W3, g2, b2)


# speedup vs baseline: 4.8922x; 1.2908x over previous
import jax
import jax.numpy as jnp
from jax.experimental import pallas as pl
from jax.experimental.pallas import tpu as pltpu

_BLK = 2048


def _ln_kernel(x_ref, w_ref, g_ref, b_ref, o_ref):
    x = x_ref[0]
    e = w_ref[0, 0]
    y = x + e[None, :]
    mean = jnp.mean(y, axis=1, keepdims=True)
    yc = y - mean
    var = jnp.mean(yc * yc, axis=1, keepdims=True)
    inv = jax.lax.rsqrt(var + 1e-9)
    o_ref[0] = yc * inv * g_ref[0][None, :] + b_ref[0][None, :]


def kernel(x, W, gamma, beta):
    B, S, D = x.shape
    W3 = W[:B].reshape(B, 1, D)
    g2 = gamma.reshape(1, D)
    b2 = beta.reshape(1, D)
    grid = (B, S // _BLK)
    return pl.pallas_call(
        _ln_kernel,
        grid=grid,
        in_specs=[
            pl.BlockSpec((1, _BLK, D), lambda b, s: (b, s, 0)),
            pl.BlockSpec((1, 1, D), lambda b, s: (b, 0, 0)),
            pl.BlockSpec((1, D), lambda b, s: (0, 0)),
            pl.BlockSpec((1, D), lambda b, s: (0, 0)),
        ],
        out_specs=pl.BlockSpec((1, _BLK, D), lambda b, s: (b, s, 0)),
        out_shape=jax.ShapeDtypeStruct((B, S, D), x.dtype),
        compiler_params=pltpu.CompilerParams(
            dimension_semantics=("parallel", "parallel"),
        ),
    )(x, W3, g2, b2)
